# Initial kernel scaffold; baseline (speedup 1.0000x reference)
import jax
import jax.numpy as jnp
from jax.experimental import pallas as pl


def kernel(h, keys_w, values_w):
    raise NotImplementedError("write your pallas kernel here")


# speedup vs baseline: 2.5428x; 2.5428x over previous
import jax
import jax.numpy as jnp
from jax.experimental import pallas as pl
from jax.experimental.pallas import tpu as pltpu

_MEMDIM = 1024
_MEMSIZE = 32768
_NHEADS = 8
_HD = _MEMDIM // _NHEADS
_BKV = 2048
_NKV = _MEMSIZE // _BKV

_LOG2E = 1.4426950408889634


def _kvmem_body(q_ref, k_ref, v_ref, o_ref, acc_ref, m_ref, l_ref):
    kv = pl.program_id(1)
    nkv = pl.num_programs(1)
    bkv = k_ref.shape[0]

    @pl.when(kv == 0)
    def _init():
        m_ref[...] = jnp.full(m_ref.shape, -jnp.inf, m_ref.dtype)
        l_ref[...] = jnp.zeros(l_ref.shape, l_ref.dtype)
        acc_ref[...] = jnp.zeros(acc_ref.shape, acc_ref.dtype)

    k = k_ref[...].astype(jnp.bfloat16)
    q = q_ref[...]
    s = jax.lax.dot_general(k, q, (((1,), (1,)), ((), ())),
                            preferred_element_type=jnp.float32)

    m_prev = m_ref[...]
    m_new = jnp.maximum(m_prev, jnp.max(s, axis=0, keepdims=True))
    alpha = jnp.exp2(m_prev - m_new)
    p = jnp.exp2(s - m_new).astype(jnp.bfloat16)

    v = v_ref[...].astype(jnp.bfloat16)
    v_aug = jnp.concatenate(
        [v, jnp.ones((16, bkv), jnp.bfloat16)], axis=0)
    pv_aug = jax.lax.dot_general(v_aug, p, (((1,), (0,)), ((), ())),
                                 preferred_element_type=jnp.float32)

    hd = v_ref.shape[0]
    m_ref[...] = m_new
    l_ref[...] = l_ref[...] * alpha + pv_aug[hd:hd + 1, :]
    acc_ref[...] = acc_ref[...] * alpha + pv_aug[:hd, :]

    @pl.when(kv == nkv - 1)
    def _finish():
        o_ref[...] = acc_ref[...] * (1.0 / l_ref[...])


def kernel(h, keys_w, values_w):
    b, s, d = h.shape
    nq = b * s
    q2 = (h.reshape(nq, d) * jnp.float32(_LOG2E)).astype(jnp.bfloat16)

    out_t = pl.pallas_call(
        _kvmem_body,
        grid=(_NHEADS, _NKV),
        in_specs=[
            pl.BlockSpec((nq, _HD), lambda hh, kv: (0, hh)),
            pl.BlockSpec((_BKV, _HD), lambda hh, kv: (kv, hh)),
            pl.BlockSpec((_HD, _BKV), lambda hh, kv: (hh, kv)),
        ],
        out_specs=pl.BlockSpec((_HD, nq), lambda hh, kv: (hh, 0)),
        out_shape=jax.ShapeDtypeStruct((d, nq), jnp.float32),
        scratch_shapes=[
            pltpu.VMEM((_HD, nq), jnp.float32),
            pltpu.VMEM((1, nq), jnp.float32),
            pltpu.VMEM((1, nq), jnp.float32),
        ],
        compiler_params=pltpu.CompilerParams(
            dimension_semantics=("parallel", "arbitrary"),
            vmem_limit_bytes=48 * 1024 * 1024,
        ),
        name="kvmem_flash",
    )(q2, keys_w, values_w)

    return out_t.T.reshape(b, s, d)


# speedup vs baseline: 2.6037x; 1.0240x over previous
import jax
import jax.numpy as jnp
from jax.experimental import pallas as pl
from jax.experimental.pallas import tpu as pltpu

_MEMDIM = 1024
_MEMSIZE = 32768
_NHEADS = 8
_HD = _MEMDIM // _NHEADS
_BKV = 2048
_NKV = _MEMSIZE // _BKV

_LOG2E = 1.4426950408889634


def _kvmem_body(q_ref, k_ref, v_ref, o_ref, acc_ref, m_ref, l_ref,
                s_buf, mb_buf):
    kv = pl.program_id(1)
    last = pl.num_programs(1) - 1
    hd = v_ref.shape[0]

    @pl.when(kv == 0)
    def _init():
        m_ref[...] = jnp.full(m_ref.shape, -jnp.inf, m_ref.dtype)
        l_ref[...] = jnp.zeros(l_ref.shape, l_ref.dtype)
        acc_ref[...] = jnp.zeros(acc_ref.shape, acc_ref.dtype)

    def stage_a(slot):
        k = k_ref[...].astype(jnp.bfloat16)
        q = q_ref[...]
        s = jax.lax.dot_general(k, q, (((1,), (1,)), ((), ())),
                                preferred_element_type=jnp.float32)
        s_buf[slot] = s
        mb_buf[slot:slot + 1, :] = jnp.max(s, axis=0, keepdims=True)

    def stage_b(slot):
        m_prev = m_ref[...]
        m_new = jnp.maximum(m_prev, mb_buf[slot:slot + 1, :])
        alpha = jnp.exp2(m_prev - m_new)
        p = jnp.exp2(s_buf[slot] - m_new).astype(jnp.bfloat16)

        v = v_ref[...].astype(jnp.bfloat16)
        v_aug = jnp.concatenate(
            [v, jnp.ones((16, v.shape[1]), jnp.bfloat16)], axis=0)
        pv = jax.lax.dot_general(v_aug, p, (((1,), (0,)), ((), ())),
                                 preferred_element_type=jnp.float32)

        m_ref[...] = m_new
        l_ref[...] = l_ref[...] * alpha + pv[hd:hd + 1, :]
        acc_ref[...] = acc_ref[...] * alpha + pv[:hd, :]

    @pl.when((kv < last) & (kv % 2 == 0))
    def _a0():
        stage_a(0)

    @pl.when((kv < last) & (kv % 2 == 1))
    def _a1():
        stage_a(1)

    @pl.when((kv > 0) & (kv % 2 == 1))
    def _b0():
        stage_b(0)

    @pl.when((kv > 0) & (kv % 2 == 0))
    def _b1():
        stage_b(1)

    @pl.when(kv == last)
    def _finish():
        o_ref[...] = acc_ref[...] * (1.0 / l_ref[...])


def kernel(h, keys_w, values_w):
    b, s, d = h.shape
    nq = b * s
    q2 = (h.reshape(nq, d) * jnp.float32(_LOG2E)).astype(jnp.bfloat16)

    nkv = _NKV

    out_t = pl.pallas_call(
        _kvmem_body,
        grid=(_NHEADS, nkv + 1),
        in_specs=[
            pl.BlockSpec((nq, _HD), lambda hh, kv: (0, hh)),
            pl.BlockSpec((_BKV, _HD),
                         lambda hh, kv: (jnp.minimum(kv, nkv - 1), hh)),
            pl.BlockSpec((_HD, _BKV),
                         lambda hh, kv: (hh, jnp.maximum(kv - 1, 0))),
        ],
        out_specs=pl.BlockSpec((_HD, nq), lambda hh, kv: (hh, 0)),
        out_shape=jax.ShapeDtypeStruct((d, nq), jnp.float32),
        scratch_shapes=[
            pltpu.VMEM((_HD, nq), jnp.float32),
            pltpu.VMEM((1, nq), jnp.float32),
            pltpu.VMEM((1, nq), jnp.float32),
            pltpu.VMEM((2, _BKV, nq), jnp.float32),
            pltpu.VMEM((2, nq), jnp.float32),
        ],
        compiler_params=pltpu.CompilerParams(
            dimension_semantics=("parallel", "arbitrary"),
            vmem_limit_bytes=56 * 1024 * 1024,
        ),
        name="kvmem_flash",
    )(q2, keys_w, values_w)

    return out_t.T.reshape(b, s, d)


# speedup vs baseline: 3.3556x; 1.2888x over previous
import jax
import jax.numpy as jnp
from jax.experimental import pallas as pl
from jax.experimental.pallas import tpu as pltpu

_MEMDIM = 1024
_MEMSIZE = 32768
_NHEADS = 8
_HD = _MEMDIM // _NHEADS
_BKV = 2048
_NKV = _MEMSIZE // _BKV

_LOG2E = 1.4426950408889634


def _kvmem_body(q_ref, qn_ref, k_ref, v_ref, o_ref, acc_ref, m_ref, l_ref):
    kv = pl.program_id(1)
    nkv = pl.num_programs(1)
    hd = v_ref.shape[0]
    bkv = k_ref.shape[0]

    @pl.when(kv == 0)
    def _init():
        m_ref[...] = jnp.zeros(m_ref.shape, m_ref.dtype)
        l_ref[...] = jnp.zeros(l_ref.shape, l_ref.dtype)
        acc_ref[...] = jnp.zeros(acc_ref.shape, acc_ref.dtype)

    kf = k_ref[...]
    k = kf.astype(jnp.bfloat16)
    q = q_ref[...]
    s = jax.lax.dot_general(k, q, (((1,), (1,)), ((), ())),
                            preferred_element_type=jnp.float32)

    msq = jnp.max(jnp.sum(kf * kf, axis=1, keepdims=True), axis=0,
                  keepdims=True)
    mb = jax.lax.sqrt(msq) * qn_ref[0]

    m_prev = m_ref[...]
    m_new = jnp.maximum(m_prev, mb)
    alpha = jnp.exp2(m_prev - m_new)
    p = jnp.exp2(s - m_new).astype(jnp.bfloat16)

    v = v_ref[...].astype(jnp.bfloat16)
    v_aug = jnp.concatenate(
        [v, jnp.ones((16, bkv), jnp.bfloat16)], axis=0)
    pv = jax.lax.dot_general(v_aug, p, (((1,), (0,)), ((), ())),
                             preferred_element_type=jnp.float32)

    m_ref[...] = m_new
    l_ref[...] = l_ref[...] * alpha + pv[hd:hd + 1, :]
    acc_ref[...] = acc_ref[...] * alpha + pv[:hd, :]

    @pl.when(kv == nkv - 1)
    def _finish():
        o_ref[...] = acc_ref[...] * (1.0 / l_ref[...])


def kernel(h, keys_w, values_w):
    b, s, d = h.shape
    nq = b * s
    hf = h.reshape(nq, d)
    q2 = (hf * jnp.float32(_LOG2E)).astype(jnp.bfloat16)
    qn = jnp.sqrt(
        jnp.sum(hf.reshape(nq, _NHEADS, _HD).astype(jnp.float32) ** 2,
                axis=-1)) * jnp.float32(_LOG2E * 1.01)
    qn = qn.T.reshape(_NHEADS, 1, nq)

    out_t = pl.pallas_call(
        _kvmem_body,
        grid=(_NHEADS, _NKV),
        in_specs=[
            pl.BlockSpec((nq, _HD), lambda hh, kv: (0, hh)),
            pl.BlockSpec((1, 1, nq), lambda hh, kv: (hh, 0, 0)),
            pl.BlockSpec((_BKV, _HD), lambda hh, kv: (kv, hh)),
            pl.BlockSpec((_HD, _BKV), lambda hh, kv: (hh, kv)),
        ],
        out_specs=pl.BlockSpec((_HD, nq), lambda hh, kv: (hh, 0)),
        out_shape=jax.ShapeDtypeStruct((d, nq), jnp.float32),
        scratch_shapes=[
            pltpu.VMEM((_HD, nq), jnp.float32),
            pltpu.VMEM((1, nq), jnp.float32),
            pltpu.VMEM((1, nq), jnp.float32),
        ],
        compiler_params=pltpu.CompilerParams(
            dimension_semantics=("parallel", "arbitrary"),
            vmem_limit_bytes=48 * 1024 * 1024,
        ),
        name="kvmem_flash",
    )(q2, qn, keys_w, values_w)

    return out_t.T.reshape(b, s, d)


# speedup vs baseline: 3.5718x; 1.0644x over previous
import jax
import jax.numpy as jnp
from jax.experimental import pallas as pl
from jax.experimental.pallas import tpu as pltpu

_MEMDIM = 1024
_MEMSIZE = 32768
_NHEADS = 8
_HD = _MEMDIM // _NHEADS
_BKV = 4096
_NKV = _MEMSIZE // _BKV

_LOG2E = 1.4426950408889634


def _kvmem_body(q_ref, qn_ref, k_ref, v_ref, o_ref, acc_ref, m_ref, l_ref):
    kv = pl.program_id(1)
    nkv = pl.num_programs(1)
    hd = v_ref.shape[0]
    bkv = k_ref.shape[0]

    @pl.when(kv == 0)
    def _init():
        m_ref[...] = jnp.zeros(m_ref.shape, m_ref.dtype)
        l_ref[...] = jnp.zeros(l_ref.shape, l_ref.dtype)
        acc_ref[...] = jnp.zeros(acc_ref.shape, acc_ref.dtype)

    kf = k_ref[...]
    k = kf.astype(jnp.bfloat16)
    q = q_ref[...]
    s = jax.lax.dot_general(k, q, (((1,), (1,)), ((), ())),
                            preferred_element_type=jnp.float32)

    msq = jnp.max(jnp.sum(kf * kf, axis=1, keepdims=True), axis=0,
                  keepdims=True)
    mb = jax.lax.sqrt(msq) * qn_ref[0]

    m_prev = m_ref[...]
    m_new = jnp.maximum(m_prev, mb)
    alpha = jnp.exp2(m_prev - m_new)
    p = jnp.exp2(s - m_new).astype(jnp.bfloat16)

    v = v_ref[...].astype(jnp.bfloat16)
    v_aug = jnp.concatenate(
        [v, jnp.ones((16, bkv), jnp.bfloat16)], axis=0)
    pv = jax.lax.dot_general(v_aug, p, (((1,), (0,)), ((), ())),
                             preferred_element_type=jnp.float32)

    m_ref[...] = m_new
    l_ref[...] = l_ref[...] * alpha + pv[hd:hd + 1, :]
    acc_ref[...] = acc_ref[...] * alpha + pv[:hd, :]

    @pl.when(kv == nkv - 1)
    def _finish():
        o_ref[...] = acc_ref[...] * (1.0 / l_ref[...])


def kernel(h, keys_w, values_w):
    b, s, d = h.shape
    nq = b * s
    hf = h.reshape(nq, d)
    q2 = (hf * jnp.float32(_LOG2E)).astype(jnp.bfloat16)
    qn = jnp.sqrt(
        jnp.sum(hf.reshape(nq, _NHEADS, _HD).astype(jnp.float32) ** 2,
                axis=-1)) * jnp.float32(_LOG2E * 1.01)
    qn = qn.T.reshape(_NHEADS, 1, nq)

    out_t = pl.pallas_call(
        _kvmem_body,
        grid=(_NHEADS, _NKV),
        in_specs=[
            pl.BlockSpec((nq, _HD), lambda hh, kv: (0, hh)),
            pl.BlockSpec((1, 1, nq), lambda hh, kv: (hh, 0, 0)),
            pl.BlockSpec((_BKV, _HD), lambda hh, kv: (kv, hh)),
            pl.BlockSpec((_HD, _BKV), lambda hh, kv: (hh, kv)),
        ],
        out_specs=pl.BlockSpec((_HD, nq), lambda hh, kv: (hh, 0)),
        out_shape=jax.ShapeDtypeStruct((d, nq), jnp.float32),
        scratch_shapes=[
            pltpu.VMEM((_HD, nq), jnp.float32),
            pltpu.VMEM((1, nq), jnp.float32),
            pltpu.VMEM((1, nq), jnp.float32),
        ],
        compiler_params=pltpu.CompilerParams(
            dimension_semantics=("parallel", "arbitrary"),
            vmem_limit_bytes=48 * 1024 * 1024,
        ),
        name="kvmem_flash",
    )(q2, qn, keys_w, values_w)

    return out_t.T.reshape(b, s, d)


# speedup vs baseline: 3.5908x; 1.0053x over previous
import jax
import jax.numpy as jnp
from jax.experimental import pallas as pl
from jax.experimental.pallas import tpu as pltpu

_MEMDIM = 1024
_MEMSIZE = 32768
_NHEADS = 8
_HD = _MEMDIM // _NHEADS
_BKV = 4096
_NKV = _MEMSIZE // _BKV

_LOG2E = 1.4426950408889634


def _kvmem_body(q_ref, qn_ref, k_ref, v_ref, o_ref, acc_ref, m_ref, l_ref):
    kv = pl.program_id(1)
    nkv = pl.num_programs(1)
    hd = v_ref.shape[0]
    bkv = k_ref.shape[0]

    @pl.when(kv == 0)
    def _init():
        m_ref[...] = jnp.zeros(m_ref.shape, m_ref.dtype)
        l_ref[...] = jnp.zeros(l_ref.shape, l_ref.dtype)
        acc_ref[...] = jnp.zeros(acc_ref.shape, acc_ref.dtype)

    kf = k_ref[...]
    q = q_ref[...]

    msq = jnp.max(jnp.sum(kf * kf, axis=1, keepdims=True), axis=0,
                  keepdims=True)
    mb = jax.lax.sqrt(msq) * qn_ref[0]

    m_prev = m_ref[...]
    m_new = jnp.maximum(m_prev, mb)
    alpha = jnp.exp2(m_prev - m_new)

    slab = min(256, bkv)
    p_slabs = []
    for i in range(bkv // slab):
        ks = kf[i * slab:(i + 1) * slab, :].astype(jnp.bfloat16)
        ss = jax.lax.dot_general(ks, q, (((1,), (1,)), ((), ())),
                                 preferred_element_type=jnp.float32)
        p_slabs.append(jnp.exp2(ss - m_new).astype(jnp.bfloat16))
    p = jnp.concatenate(p_slabs, axis=0)

    v = v_ref[...].astype(jnp.bfloat16)
    v_aug = jnp.concatenate(
        [v, jnp.ones((16, bkv), jnp.bfloat16)], axis=0)
    pv = jax.lax.dot_general(v_aug, p, (((1,), (0,)), ((), ())),
                             preferred_element_type=jnp.float32)

    m_ref[...] = m_new
    l_ref[...] = l_ref[...] * alpha + pv[hd:hd + 1, :]
    acc_ref[...] = acc_ref[...] * alpha + pv[:hd, :]

    @pl.when(kv == nkv - 1)
    def _finish():
        o_ref[...] = acc_ref[...] * (1.0 / l_ref[...])


def kernel(h, keys_w, values_w):
    b, s, d = h.shape
    nq = b * s
    hf = h.reshape(nq, d)
    q2 = (hf * jnp.float32(_LOG2E)).astype(jnp.bfloat16)
    qn = jnp.sqrt(
        jnp.sum(hf.reshape(nq, _NHEADS, _HD).astype(jnp.float32) ** 2,
                axis=-1)) * jnp.float32(_LOG2E * 1.01)
    qn = qn.T.reshape(_NHEADS, 1, nq)

    out_t = pl.pallas_call(
        _kvmem_body,
        grid=(_NHEADS, _NKV),
        in_specs=[
            pl.BlockSpec((nq, _HD), lambda hh, kv: (0, hh)),
            pl.BlockSpec((1, 1, nq), lambda hh, kv: (hh, 0, 0)),
            pl.BlockSpec((_BKV, _HD), lambda hh, kv: (kv, hh)),
            pl.BlockSpec((_HD, _BKV), lambda hh, kv: (hh, kv)),
        ],
        out_specs=pl.BlockSpec((_HD, nq), lambda hh, kv: (hh, 0)),
        out_shape=jax.ShapeDtypeStruct((d, nq), jnp.float32),
        scratch_shapes=[
            pltpu.VMEM((_HD, nq), jnp.float32),
            pltpu.VMEM((1, nq), jnp.float32),
            pltpu.VMEM((1, nq), jnp.float32),
        ],
        compiler_params=pltpu.CompilerParams(
            dimension_semantics=("parallel", "arbitrary"),
            vmem_limit_bytes=48 * 1024 * 1024,
        ),
        name="kvmem_flash",
    )(q2, qn, keys_w, values_w)

    return out_t.T.reshape(b, s, d)


# speedup vs baseline: 3.6476x; 1.0158x over previous
import jax
import jax.numpy as jnp
from jax.experimental import pallas as pl
from jax.experimental.pallas import tpu as pltpu

_MEMDIM = 1024
_MEMSIZE = 32768
_NHEADS = 8
_HD = _MEMDIM // _NHEADS
_BKV = 4096
_NKV = _MEMSIZE // _BKV

_LOG2E = 1.4426950408889634


def _kvmem_body(q_ref, qn_ref, k_ref, v_ref, o_ref, acc_ref, m_ref, l_ref):
    kv = pl.program_id(1)
    nkv = pl.num_programs(1)
    hd = v_ref.shape[0]
    bkv = k_ref.shape[0]

    @pl.when(kv == 0)
    def _init():
        m_ref[...] = jnp.zeros(m_ref.shape, m_ref.dtype)
        l_ref[...] = jnp.zeros(l_ref.shape, l_ref.dtype)
        acc_ref[...] = jnp.zeros(acc_ref.shape, acc_ref.dtype)

    kf = k_ref[...]
    q = q_ref[...]

    msq = jnp.max(jnp.sum(kf * kf, axis=1, keepdims=True), axis=0,
                  keepdims=True)
    mb = jax.lax.sqrt(msq) * qn_ref[0]

    m_prev = m_ref[...]
    m_new = jnp.maximum(m_prev, mb)
    alpha = jnp.exp2(m_prev - m_new)

    slab = min(256, bkv)
    p_slabs = []
    for i in range(bkv // slab):
        ks = kf[i * slab:(i + 1) * slab, :].astype(jnp.bfloat16)
        ss = jax.lax.dot_general(ks, q, (((1,), (1,)), ((), ())),
                                 preferred_element_type=jnp.float32)
        p_slabs.append(jnp.exp2(ss - m_new).astype(jnp.bfloat16))
    p = jnp.concatenate(p_slabs, axis=0)

    v = v_ref[...].astype(jnp.bfloat16)
    v_aug = jnp.concatenate(
        [v, jnp.ones((16, bkv), jnp.bfloat16)], axis=0)
    pv = jax.lax.dot_general(v_aug, p, (((1,), (0,)), ((), ())),
                             preferred_element_type=jnp.float32)

    m_ref[...] = m_new
    l_ref[...] = l_ref[...] * alpha + pv[hd:hd + 1, :]
    acc_ref[...] = acc_ref[...] * alpha + pv[:hd, :]

    @pl.when(kv == nkv - 1)
    def _finish():
        o_ref[...] = (acc_ref[...] * (1.0 / l_ref[...])).T


def kernel(h, keys_w, values_w):
    b, s, d = h.shape
    nq = b * s
    hf = h.reshape(nq, d)
    q2 = (hf * jnp.float32(_LOG2E)).astype(jnp.bfloat16)
    qn = jnp.sqrt(
        jnp.sum(hf.reshape(nq, _NHEADS, _HD).astype(jnp.float32) ** 2,
                axis=-1)) * jnp.float32(_LOG2E * 1.01)
    qn = qn.T.reshape(_NHEADS, 1, nq)

    out_t = pl.pallas_call(
        _kvmem_body,
        grid=(_NHEADS, _NKV),
        in_specs=[
            pl.BlockSpec((nq, _HD), lambda hh, kv: (0, hh)),
            pl.BlockSpec((1, 1, nq), lambda hh, kv: (hh, 0, 0)),
            pl.BlockSpec((_BKV, _HD), lambda hh, kv: (kv, hh)),
            pl.BlockSpec((_HD, _BKV), lambda hh, kv: (hh, kv)),
        ],
        out_specs=pl.BlockSpec((nq, _HD), lambda hh, kv: (0, hh)),
        out_shape=jax.ShapeDtypeStruct((nq, d), jnp.float32),
        scratch_shapes=[
            pltpu.VMEM((_HD, nq), jnp.float32),
            pltpu.VMEM((1, nq), jnp.float32),
            pltpu.VMEM((1, nq), jnp.float32),
        ],
        compiler_params=pltpu.CompilerParams(
            dimension_semantics=("parallel", "arbitrary"),
            vmem_limit_bytes=48 * 1024 * 1024,
        ),
        name="kvmem_flash",
    )(q2, qn, keys_w, values_w)

    return out_t.reshape(b, s, d)
